# baseline (speedup 1.0000x reference)
import functools

import jax
import jax.numpy as jnp
from jax import lax
from jax.experimental import pallas as pl
from jax.experimental.pallas import tpu as pltpu
from jax.experimental.pallas import tpu_sc as plsc

_VOCAB = 1000000
_D = 32
_BATCH = 16384
_FIELDS = 26
_B_TOTAL = _BATCH * _FIELDS
_NC, _NS = 2, 16
_NW = _NC * _NS
_BPW = _B_TOTAL // _NW
_CHUNK = 1024
_N_CHUNKS = _BPW // _CHUNK

_BR = 2048
_BC = 2048


@functools.cache
def _table_to_rowmajor():
    def body(wt_ref, out_ref):
        out_ref[...] = wt_ref[...].T

    return pl.pallas_call(
        body,
        grid=(pl.cdiv(_VOCAB, _BR),),
        in_specs=[pl.BlockSpec((_D, _BR), lambda i: (0, i))],
        out_specs=pl.BlockSpec((_BR, _D), lambda i: (i, 0)),
        out_shape=jax.ShapeDtypeStruct((_VOCAB, _D), jnp.float32),
    )


@functools.cache
def _rows_to_native():
    nj = _BATCH // _BC

    def body(r_ref, out_ref):
        out_ref[0] = r_ref[...].T

    return pl.pallas_call(
        body,
        grid=(_FIELDS, nj),
        in_specs=[pl.BlockSpec((_BC, _D), lambda f, j: (f * nj + j, 0))],
        out_specs=pl.BlockSpec((1, _D, _BC), lambda f, j: (f, 0, j)),
        out_shape=jax.ShapeDtypeStruct((_FIELDS, _D, _BATCH), jnp.float32),
    )


@functools.cache
def _sc_gather():
    mesh = plsc.VectorSubcoreMesh(
        core_axis_name="c", subcore_axis_name="s",
        num_cores=_NC, num_subcores=_NS,
    )

    @functools.partial(
        pl.kernel,
        out_type=jax.ShapeDtypeStruct((_B_TOTAL, _D), jnp.float32),
        mesh=mesh,
        compiler_params=pltpu.CompilerParams(use_tc_tiling_on_sc=False),
        scratch_types=[
            pltpu.VMEM((_BPW,), jnp.int32),
            pltpu.VMEM((_CHUNK, _D), jnp.float32),
            pltpu.VMEM((_CHUNK, _D), jnp.float32),
            pltpu.SemaphoreType.DMA,
            pltpu.SemaphoreType.DMA,
            pltpu.SemaphoreType.DMA,
            pltpu.SemaphoreType.DMA,
        ],
    )
    def lookup(idx_hbm, table_hbm, out_hbm,
               idx_v, rows0, rows1, gsem0, gsem1, osem0, osem1):
        wid = lax.axis_index("s") * _NC + lax.axis_index("c")
        base = wid * _BPW
        pltpu.sync_copy(idx_hbm.at[pl.ds(base, _BPW)], idx_v)

        rows = (rows0, rows1)
        gsem = (gsem0, gsem1)
        osem = (osem0, osem1)
        gcp = [None, None]
        ocp = [None, None]
        for i in range(_N_CHUNKS):
            b = i & 1
            if ocp[b] is not None:
                ocp[b].wait()
            gcp[b] = pltpu.async_copy(
                table_hbm.at[idx_v.at[pl.ds(i * _CHUNK, _CHUNK)]],
                rows[b], gsem[b])
            if i > 0:
                pb = (i - 1) & 1
                gcp[pb].wait()
                ocp[pb] = pltpu.async_copy(
                    rows[pb],
                    out_hbm.at[pl.ds(base + (i - 1) * _CHUNK, _CHUNK)],
                    osem[pb])
        last = (_N_CHUNKS - 1) & 1
        gcp[last].wait()
        ocp[last] = pltpu.async_copy(
            rows[last],
            out_hbm.at[pl.ds(base + (_N_CHUNKS - 1) * _CHUNK, _CHUNK)],
            osem[last])
        ocp[1 - last].wait()
        ocp[last].wait()

    return lookup


@jax.jit
def kernel(input, weight):
    idx = input.T.reshape(-1).astype(jnp.int32)
    w_rm = _table_to_rowmajor()(weight.T)
    rows = _sc_gather()(idx, w_rm)
    out3 = _rows_to_native()(rows)
    return out3.transpose(2, 0, 1)


# speedup vs baseline: 1.3061x; 1.3061x over previous
import functools

import jax
import jax.numpy as jnp
from jax import lax
from jax.experimental import pallas as pl
from jax.experimental.pallas import tpu as pltpu
from jax.experimental.pallas import tpu_sc as plsc

_VOCAB = 1000000
_D = 32
_BATCH = 16384
_FIELDS = 26
_B_TOTAL = _BATCH * _FIELDS
_NC, _NS = 2, 16
_NW = _NC * _NS
_BPW = _B_TOTAL // _NW
_CHUNK = 1024
_N_CHUNKS = _BPW // _CHUNK

_Q = 1 << 18
_BR = 8192
_NBLK = _Q // _BR


@functools.cache
def _table_to_rowmajor():
    def body(wt_ref, out_ref):
        out_ref[...] = wt_ref[...].T

    return pl.pallas_call(
        body,
        grid=(pl.cdiv(_VOCAB, _BR),),
        in_specs=[pl.BlockSpec((_D, _BR), lambda i: (0, i))],
        out_specs=pl.BlockSpec((_BR, _D), lambda i: (i, 0)),
        out_shape=jax.ShapeDtypeStruct((_VOCAB, _D), jnp.float32),
    )


@functools.cache
def _sc_gather():
    mesh = plsc.VectorSubcoreMesh(
        core_axis_name="c", subcore_axis_name="s",
        num_cores=_NC, num_subcores=_NS,
    )

    @functools.partial(
        pl.kernel,
        out_type=jax.ShapeDtypeStruct((_B_TOTAL, _D), jnp.float32),
        mesh=mesh,
        compiler_params=pltpu.CompilerParams(use_tc_tiling_on_sc=False),
        scratch_types=[
            pltpu.VMEM((_BPW,), jnp.int32),
            pltpu.VMEM((_CHUNK, _D), jnp.float32),
            pltpu.VMEM((_CHUNK, _D), jnp.float32),
            pltpu.SemaphoreType.DMA,
            pltpu.SemaphoreType.DMA,
            pltpu.SemaphoreType.DMA,
            pltpu.SemaphoreType.DMA,
        ],
    )
    def lookup(idx_hbm, table_hbm, out_hbm,
               idx_v, rows0, rows1, gsem0, gsem1, osem0, osem1):
        wid = lax.axis_index("s") * _NC + lax.axis_index("c")
        base = wid * _BPW
        pltpu.sync_copy(idx_hbm.at[pl.ds(base, _BPW)], idx_v)

        rows = (rows0, rows1)
        gsem = (gsem0, gsem1)
        osem = (osem0, osem1)
        gcp = [None, None]
        ocp = [None, None]
        for i in range(_N_CHUNKS):
            b = i & 1
            if ocp[b] is not None:
                ocp[b].wait()
            gcp[b] = pltpu.async_copy(
                table_hbm.at[idx_v.at[pl.ds(i * _CHUNK, _CHUNK)]],
                rows[b], gsem[b])
            if i > 0:
                pb = (i - 1) & 1
                gcp[pb].wait()
                ocp[pb] = pltpu.async_copy(
                    rows[pb],
                    out_hbm.at[pl.ds(base + (i - 1) * _CHUNK, _CHUNK)],
                    osem[pb])
        last = (_N_CHUNKS - 1) & 1
        gcp[last].wait()
        ocp[last] = pltpu.async_copy(
            rows[last],
            out_hbm.at[pl.ds(base + (_N_CHUNKS - 1) * _CHUNK, _CHUNK)],
            osem[last])
        ocp[1 - last].wait()
        ocp[last].wait()

    return lookup


@jax.jit
def kernel(input, weight):
    idx = input.reshape(-1).astype(jnp.int32)
    w_rm = _table_to_rowmajor()(weight.T)
    rows = _sc_gather()(idx, w_rm)
    return rows.reshape(input.shape + (weight.shape[1],))


# speedup vs baseline: 1.4553x; 1.1143x over previous
import functools

import jax
import jax.numpy as jnp
from jax import lax
from jax.experimental import pallas as pl
from jax.experimental.pallas import tpu as pltpu
from jax.experimental.pallas import tpu_sc as plsc

_VOCAB = 1000000
_D = 32
_B_TOTAL = 16384 * 26
_NC, _NS = 2, 16
_NW = _NC * _NS
_BPW = _B_TOTAL // _NW
_CHUNK = 1024
_N_CHUNKS = _BPW // _CHUNK


@functools.cache
def _make_lookup():
    mesh = plsc.VectorSubcoreMesh(
        core_axis_name="c", subcore_axis_name="s",
        num_cores=_NC, num_subcores=_NS,
    )

    @functools.partial(
        pl.kernel,
        out_type=jax.ShapeDtypeStruct((_B_TOTAL, _D), jnp.float32),
        mesh=mesh,
        compiler_params=pltpu.CompilerParams(use_tc_tiling_on_sc=False),
        scratch_types=[
            pltpu.VMEM((_BPW,), jnp.int32),
            pltpu.VMEM((_CHUNK, _D), jnp.float32),
            pltpu.VMEM((_CHUNK, _D), jnp.float32),
            pltpu.SemaphoreType.DMA,
            pltpu.SemaphoreType.DMA,
            pltpu.SemaphoreType.DMA,
            pltpu.SemaphoreType.DMA,
        ],
    )
    def lookup(idx_hbm, table_hbm, out_hbm,
               idx_v, rows0, rows1, gsem0, gsem1, osem0, osem1):
        wid = lax.axis_index("s") * _NC + lax.axis_index("c")
        base = wid * _BPW
        pltpu.sync_copy(idx_hbm.at[pl.ds(base, _BPW)], idx_v)

        rows = (rows0, rows1)
        gsem = (gsem0, gsem1)
        osem = (osem0, osem1)
        gcp = [None, None]
        ocp = [None, None]
        for i in range(_N_CHUNKS):
            b = i & 1
            if ocp[b] is not None:
                ocp[b].wait()
            gcp[b] = pltpu.async_copy(
                table_hbm.at[idx_v.at[pl.ds(i * _CHUNK, _CHUNK)]],
                rows[b], gsem[b])
            if i > 0:
                pb = (i - 1) & 1
                gcp[pb].wait()
                ocp[pb] = pltpu.async_copy(
                    rows[pb],
                    out_hbm.at[pl.ds(base + (i - 1) * _CHUNK, _CHUNK)],
                    osem[pb])
        last = (_N_CHUNKS - 1) & 1
        gcp[last].wait()
        ocp[last] = pltpu.async_copy(
            rows[last],
            out_hbm.at[pl.ds(base + (_N_CHUNKS - 1) * _CHUNK, _CHUNK)],
            osem[last])
        ocp[1 - last].wait()
        ocp[last].wait()

    return lookup


@jax.jit
def kernel(input, weight):
    idx = input.reshape(-1).astype(jnp.int32)
    out = _make_lookup()(idx, weight)
    return out.reshape(input.shape + (weight.shape[1],))


# speedup vs baseline: 2.5324x; 1.7401x over previous
import functools

import jax
import jax.numpy as jnp
from jax import lax
from jax.experimental import pallas as pl
from jax.experimental.pallas import tpu as pltpu
from jax.experimental.pallas import tpu_sc as plsc

_VOCAB = 1000000
_D = 32
_B_TOTAL = 16384 * 26
_NC, _NS = 2, 16
_NW = _NC * _NS
_BPW = _B_TOTAL // _NW
_CHUNK = 1024
_N_CHUNKS = _BPW // _CHUNK
_Q = 1 << 18
_BR = 2048
_NBLK = _Q // _BR


@functools.cache
def _table_to_rowmajor():
    def body(x0, x1, x2, x3, out_ref):
        x4 = jnp.concatenate([x0[...], x1[...], x2[...], x3[...]], axis=0)
        out_ref[...] = x4.T

    def spec(s):
        return pl.BlockSpec(
            (_D, _BR),
            lambda i, s=s: (0, jnp.minimum(s * _NBLK + i, _VOCAB // _BR)))

    return pl.pallas_call(
        body,
        grid=(_NBLK,),
        in_specs=[spec(0), spec(1), spec(2), spec(3)],
        out_specs=pl.BlockSpec((_BR, 128), lambda i: (i, 0)),
        out_shape=jax.ShapeDtypeStruct((_Q, 128), jnp.float32),
    )


@functools.cache
def _make_lookup():
    mesh = plsc.VectorSubcoreMesh(
        core_axis_name="c", subcore_axis_name="s",
        num_cores=_NC, num_subcores=_NS,
    )

    @functools.partial(
        pl.kernel,
        out_type=jax.ShapeDtypeStruct((_B_TOTAL, _D), jnp.float32),
        mesh=mesh,
        compiler_params=pltpu.CompilerParams(use_tc_tiling_on_sc=False),
        scratch_types=[
            pltpu.VMEM((_BPW,), jnp.int32),
            pltpu.VMEM((_CHUNK, _D), jnp.float32),
            pltpu.VMEM((_CHUNK, _D), jnp.float32),
            pltpu.SemaphoreType.DMA,
            pltpu.SemaphoreType.DMA,
            pltpu.SemaphoreType.DMA,
            pltpu.SemaphoreType.DMA,
        ],
    )
    def lookup(idx_hbm, table_hbm, out_hbm,
               idx_v, rows0, rows1, gsem0, gsem1, osem0, osem1):
        wid = lax.axis_index("s") * _NC + lax.axis_index("c")
        base = wid * _BPW
        pltpu.sync_copy(idx_hbm.at[pl.ds(base, _BPW)], idx_v)

        rows = (rows0, rows1)
        gsem = (gsem0, gsem1)
        osem = (osem0, osem1)
        gcp = [None, None]
        ocp = [None, None]
        for i in range(_N_CHUNKS):
            b = i & 1
            if ocp[b] is not None:
                ocp[b].wait()
            gcp[b] = pltpu.async_copy(
                table_hbm.at[idx_v.at[pl.ds(i * _CHUNK, _CHUNK)]],
                rows[b], gsem[b])
            if i > 0:
                pb = (i - 1) & 1
                gcp[pb].wait()
                ocp[pb] = pltpu.async_copy(
                    rows[pb],
                    out_hbm.at[pl.ds(base + (i - 1) * _CHUNK, _CHUNK)],
                    osem[pb])
        last = (_N_CHUNKS - 1) & 1
        gcp[last].wait()
        ocp[last] = pltpu.async_copy(
            rows[last],
            out_hbm.at[pl.ds(base + (_N_CHUNKS - 1) * _CHUNK, _CHUNK)],
            osem[last])
        ocp[1 - last].wait()
        ocp[last].wait()

    return lookup


@jax.jit
def kernel(input, weight):
    r = input.reshape(-1).astype(jnp.int32)
    idx = ((r & (_Q - 1)) << 2) | (r >> 18)
    wt = weight.T
    w2 = _table_to_rowmajor()(wt, wt, wt, wt)
    w_rm = w2.reshape(4 * _Q, _D)
    out = _make_lookup()(idx, w_rm)
    return out.reshape(input.shape + (weight.shape[1],))


# speedup vs baseline: 3.6449x; 1.4393x over previous
import functools

import jax
import jax.numpy as jnp
from jax import lax
from jax.experimental import pallas as pl
from jax.experimental.pallas import tpu as pltpu
from jax.experimental.pallas import tpu_sc as plsc

_VOCAB = 1000000
_D = 32
_B_TOTAL = 16384 * 26
_NC, _NS = 2, 16
_NW = _NC * _NS
_BPW = _B_TOTAL // _NW
_CHUNK = 1024
_N_CHUNKS = _BPW // _CHUNK
_Q = 1 << 18
_BR = 2048
_NBLK = _Q // _BR


@functools.cache
def _table_to_rowmajor():
    def body(x0, x1, x2, x3, out_ref):
        x4 = jnp.concatenate([x0[...], x1[...], x2[...], x3[...]], axis=0)
        out_ref[...] = x4.T

    def spec(s):
        return pl.BlockSpec(
            (_D, _BR),
            lambda i, s=s: (0, jnp.minimum(s * _NBLK + i, _VOCAB // _BR)))

    return pl.pallas_call(
        body,
        grid=(_NBLK,),
        in_specs=[spec(0), spec(1), spec(2), spec(3)],
        out_specs=pl.BlockSpec((_BR, 128), lambda i: (i, 0)),
        out_shape=jax.ShapeDtypeStruct((_Q, 128), jnp.float32),
    )


@functools.cache
def _rows_to_native():
    def body(r_ref, out_ref):
        y = r_ref[...].T
        for u in range(4):
            for dg in range(4):
                for bgl in range(32):
                    out_ref[0, dg, u * 32 + bgl] = jax.lax.slice(
                        y, (u * 32 + dg * 8, bgl * 128),
                        (u * 32 + dg * 8 + 8, bgl * 128 + 128))

    return pl.pallas_call(
        body,
        grid=(26,),
        in_specs=[pl.BlockSpec((4096, 128), lambda f: (f, 0))],
        out_specs=pl.BlockSpec((1, 4, 128, 8, 128), lambda f: (f, 0, 0, 0, 0)),
        out_shape=jax.ShapeDtypeStruct((26, 4, 128, 8, 128), jnp.float32),
    )


@functools.cache
def _make_lookup():
    mesh = plsc.VectorSubcoreMesh(
        core_axis_name="c", subcore_axis_name="s",
        num_cores=_NC, num_subcores=_NS,
    )

    @functools.partial(
        pl.kernel,
        out_type=jax.ShapeDtypeStruct((_B_TOTAL, _D), jnp.float32),
        mesh=mesh,
        compiler_params=pltpu.CompilerParams(use_tc_tiling_on_sc=False),
        scratch_types=[
            pltpu.VMEM((_BPW,), jnp.int32),
            pltpu.VMEM((_CHUNK, _D), jnp.float32),
            pltpu.VMEM((_CHUNK, _D), jnp.float32),
            pltpu.SemaphoreType.DMA,
            pltpu.SemaphoreType.DMA,
            pltpu.SemaphoreType.DMA,
            pltpu.SemaphoreType.DMA,
        ],
    )
    def lookup(idx_hbm, table_hbm, out_hbm,
               idx_v, rows0, rows1, gsem0, gsem1, osem0, osem1):
        wid = lax.axis_index("s") * _NC + lax.axis_index("c")
        base = wid * _BPW
        pltpu.sync_copy(idx_hbm.at[pl.ds(base, _BPW)], idx_v)

        rows = (rows0, rows1)
        gsem = (gsem0, gsem1)
        osem = (osem0, osem1)
        gcp = [None, None]
        ocp = [None, None]
        for i in range(_N_CHUNKS):
            b = i & 1
            if ocp[b] is not None:
                ocp[b].wait()
            gcp[b] = pltpu.async_copy(
                table_hbm.at[idx_v.at[pl.ds(i * _CHUNK, _CHUNK)]],
                rows[b], gsem[b])
            if i > 0:
                pb = (i - 1) & 1
                gcp[pb].wait()
                ocp[pb] = pltpu.async_copy(
                    rows[pb],
                    out_hbm.at[pl.ds(base + (i - 1) * _CHUNK, _CHUNK)],
                    osem[pb])
        last = (_N_CHUNKS - 1) & 1
        gcp[last].wait()
        ocp[last] = pltpu.async_copy(
            rows[last],
            out_hbm.at[pl.ds(base + (_N_CHUNKS - 1) * _CHUNK, _CHUNK)],
            osem[last])
        ocp[1 - last].wait()
        ocp[last].wait()

    return lookup


@jax.jit
def kernel(input, weight):
    r = (input.T.astype(jnp.int32)
         .reshape(26, 4, 4096).transpose(0, 2, 1).reshape(-1))
    idx = ((r & (_Q - 1)) << 2) | (r >> 18)
    wt = weight.T
    w2 = _table_to_rowmajor()(wt, wt, wt, wt)
    w_rm = w2.reshape(4 * _Q, _D)
    rows = _make_lookup()(idx, w_rm)
    rows128 = rows.reshape(26 * 4096, 128)
    out6 = _rows_to_native()(rows128)
    o1 = out6.transpose(0, 1, 3, 2, 4)
    o2 = o1.reshape(26, _D, 16384)
    return o2.transpose(2, 0, 1)
